# Initial kernel scaffold; baseline (speedup 1.0000x reference)
import jax
import jax.numpy as jnp
from jax.experimental import pallas as pl


def kernel(x, edge_index, edge_weight, W1, b1, W2, b2):
    raise NotImplementedError("write your pallas kernel here")


# speedup vs baseline: 40.5750x; 40.5750x over previous
import functools

import jax
import jax.numpy as jnp
from jax import lax
from jax.experimental import pallas as pl
from jax.experimental.pallas import tpu as pltpu
from jax.experimental.pallas import tpu_sc as plsc

NC = 2
NS = 16
NT = NC * NS
CB = 128
KB = 8
C = CB * KB

f32 = jnp.float32
i32 = jnp.int32


def _make_sc_kernels(Np, NCH):
    SL = Np // NS
    RPT = NCH * KB
    mesh = plsc.VectorSubcoreMesh(core_axis_name="c", subcore_axis_name="s")

    def deg_body(col_hbm, w_hbm, zer1_hbm, out_hbm, col_v, w_v, acc_sh):
        cid = lax.axis_index("c")
        sid = lax.axis_index("s")
        base = (cid * NS + sid) * RPT
        pltpu.sync_copy(zer1_hbm, acc_sh.at[pl.ds(sid * SL, SL)])
        plsc.subcore_barrier()

        def chunk(j, carry):
            off = base + j * KB
            pltpu.sync_copy(col_hbm.at[pl.ds(off, KB)], col_v)
            pltpu.sync_copy(w_hbm.at[pl.ds(off, KB)], w_v)
            for k in range(KB):
                pltpu.sync_copy(w_v.at[k], acc_sh.at[col_v.at[k]], add=True)
            return carry

        lax.fori_loop(0, NCH, chunk, 0)
        plsc.subcore_barrier()
        tid = cid * NS + sid
        pltpu.sync_copy(acc_sh.at[pl.ds(sid * SL, SL)],
                        out_hbm.at[pl.ds(tid * SL, SL)])

    deg_call = pl.kernel(
        deg_body,
        out_type=jax.ShapeDtypeStruct((NC * Np,), f32),
        mesh=mesh,
        scratch_types=[
            pltpu.VMEM((KB, CB), i32),
            pltpu.VMEM((KB, CB), f32),
            pltpu.VMEM_SHARED((Np,), f32),
        ],
    )

    def agg1_body(row_hbm, col_hbm, w_hbm, y_hbm, zer2_hbm, out_hbm,
                  row_v, col_v, w_v, rows_v, acc_sh, sem):
        cid = lax.axis_index("c")
        sid = lax.axis_index("s")
        base = (cid * NS + sid) * RPT
        pltpu.sync_copy(zer2_hbm, acc_sh.at[pl.ds(sid * SL, SL)])
        plsc.subcore_barrier()

        def chunk(j, carry):
            off = base + j * KB
            pltpu.sync_copy(row_hbm.at[pl.ds(off, KB)], row_v)
            pltpu.sync_copy(col_hbm.at[pl.ds(off, KB)], col_v)
            pltpu.sync_copy(w_hbm.at[pl.ds(off, KB)], w_v)
            descs = [pltpu.async_copy(y_hbm.at[row_v.at[k]], rows_v.at[k], sem)
                     for k in range(KB)]
            for d in descs:
                d.wait()
            for k in range(KB):
                def scale(i, carry2, k=k):
                    wv = w_v[k, pl.ds(i * 16, 16)]
                    for l in range(16):
                        e = i * 16 + l
                        rows_v[k, e] = rows_v[k, e] * wv[l]
                    return carry2
                lax.fori_loop(0, CB // 16, scale, 0)
            for k in range(KB):
                pltpu.sync_copy(rows_v.at[k], acc_sh.at[col_v.at[k]], add=True)
            return carry

        lax.fori_loop(0, NCH, chunk, 0)
        plsc.subcore_barrier()
        tid = cid * NS + sid
        pltpu.sync_copy(acc_sh.at[pl.ds(sid * SL, SL)], out_hbm.at[tid])

    agg1_call = pl.kernel(
        agg1_body,
        out_type=jax.ShapeDtypeStruct((NT, SL, 16), f32),
        mesh=mesh,
        compiler_params=pltpu.CompilerParams(use_tc_tiling_on_sc=False),
        scratch_types=[
            pltpu.VMEM((KB, CB), i32),
            pltpu.VMEM((KB, CB), i32),
            pltpu.VMEM((KB, CB), f32),
            pltpu.VMEM((KB, CB, 16), f32),
            pltpu.VMEM_SHARED((Np, 16), f32),
            pltpu.SemaphoreType.DMA,
        ],
    )

    def agg2_body(rowf_hbm, col_hbm, wf_hbm, zz_hbm, zer1_hbm, out_hbm,
                  row_v, col_v, w_v, msg_v, zz_v, acc_sh):
        cid = lax.axis_index("c")
        sid = lax.axis_index("s")
        base = (cid * NS + sid) * RPT
        pltpu.sync_copy(zz_hbm, zz_v)
        pltpu.sync_copy(zer1_hbm, acc_sh.at[pl.ds(sid * SL, SL)])
        plsc.subcore_barrier()

        def chunk(j, carry):
            off = base + j * KB
            offe = off * CB
            pltpu.sync_copy(rowf_hbm.at[pl.ds(offe, C)], row_v)
            pltpu.sync_copy(col_hbm.at[pl.ds(off, KB)], col_v)
            pltpu.sync_copy(wf_hbm.at[pl.ds(offe, C)], w_v)

            def gat(i, carry2):
                idx = row_v[pl.ds(i * 16, 16)]
                vals = plsc.load_gather(zz_v, [idx])
                msg_v[pl.ds(i * 16, 16)] = vals * w_v[pl.ds(i * 16, 16)]
                return carry2

            lax.fori_loop(0, C // 16, gat, 0)
            for k in range(KB):
                pltpu.sync_copy(msg_v.at[pl.ds(k * CB, CB)],
                                acc_sh.at[col_v.at[k]], add=True)
            return carry

        lax.fori_loop(0, NCH, chunk, 0)
        plsc.subcore_barrier()
        tid = cid * NS + sid
        pltpu.sync_copy(acc_sh.at[pl.ds(sid * SL, SL)],
                        out_hbm.at[pl.ds(tid * SL, SL)])

    agg2_call = pl.kernel(
        agg2_body,
        out_type=jax.ShapeDtypeStruct((NC * Np,), f32),
        mesh=mesh,
        compiler_params=pltpu.CompilerParams(needs_layout_passes=False),
        scratch_types=[
            pltpu.VMEM((C,), i32),
            pltpu.VMEM((KB, CB), i32),
            pltpu.VMEM((C,), f32),
            pltpu.VMEM((C,), f32),
            pltpu.VMEM((Np,), f32),
            pltpu.VMEM_SHARED((Np,), f32),
        ],
    )

    return deg_call, agg1_call, agg2_call


def _tc_prep(degp3, xp3, R):
    def body(degp_ref, xp_ref, dis_ref, y_ref):
        deg = degp_ref[0] + degp_ref[1] + 1.0
        dis = lax.rsqrt(deg)
        dis_ref[...] = dis
        y_ref[...] = xp_ref[...] * dis[:, :, None]

    return pl.pallas_call(
        body,
        out_shape=(jax.ShapeDtypeStruct((R, 128), f32),
                   jax.ShapeDtypeStruct((R, 128, 16), f32)),
    )(degp3, xp3)


def _tc_dense(acc0, acc1, y, dis1, W1p, b1r, W2, Np):
    GB = Np // 8
    grid = Np // GB

    def body(a0, a1, yv, dv, w1, b1v, w2, zzv):
        A = (a0[...] + a1[...] + yv[...]) * dv[...]
        h = jnp.maximum(
            jnp.dot(A, w1[...], preferred_element_type=f32) + b1v[...], 0.0)
        zzv[...] = jnp.dot(h, w2[...], preferred_element_type=f32) * dv[...]

    return pl.pallas_call(
        body,
        grid=(grid,),
        in_specs=[
            pl.BlockSpec((GB, 16), lambda i: (i, 0)),
            pl.BlockSpec((GB, 16), lambda i: (i, 0)),
            pl.BlockSpec((GB, 16), lambda i: (i, 0)),
            pl.BlockSpec((GB, 1), lambda i: (i, 0)),
            pl.BlockSpec((16, 64), lambda i: (0, 0)),
            pl.BlockSpec((1, 64), lambda i: (0, 0)),
            pl.BlockSpec((64, 1), lambda i: (0, 0)),
        ],
        out_specs=pl.BlockSpec((GB, 1), lambda i: (i, 0)),
        out_shape=jax.ShapeDtypeStruct((Np, 1), f32),
    )(acc0, acc1, y, dis1, W1p, b1r, W2)


def _tc_final(acc2p3, zz2, dis2, b2r):
    def body(ap, zzr, dr, br, outr):
        outr[...] = (ap[0] + ap[1] + zzr[...]) * dr[...] + br[0, 0]

    return pl.pallas_call(
        body,
        out_shape=jax.ShapeDtypeStruct(dis2.shape, f32),
    )(acc2p3, zz2, dis2, b2r)


def kernel(x, edge_index, edge_weight, W1, b1, W2, b2):
    N, D = x.shape
    H = W1.shape[1]
    E = edge_index.shape[1]
    Dp = 16
    Np = -(-N // 2048) * 2048
    R = Np // 128
    Ep = -(-E // (NT * C)) * NT * C
    NCH = Ep // (NT * C)
    EpR = Ep // CB

    row = edge_index[0].astype(i32)
    col = edge_index[1].astype(i32)
    w = edge_weight.astype(f32)
    padE = Ep - E
    row2 = jnp.pad(row, (0, padE)).reshape(EpR, CB)
    col2 = jnp.pad(col, (0, padE)).reshape(EpR, CB)
    w2 = jnp.pad(w, (0, padE)).reshape(EpR, CB)
    xp3 = jnp.pad(x, ((0, Np - N), (0, Dp - D))).reshape(R, 128, Dp)
    zer1 = jnp.zeros((Np // NS,), f32)
    zer2 = jnp.zeros((Np // NS, Dp), f32)
    W1p = jnp.pad(W1, ((0, Dp - D), (0, 0)))
    b1r = b1.reshape(1, H)
    b2r = b2.reshape(1, 1)

    deg_call, agg1_call, agg2_call = _make_sc_kernels(Np, NCH)

    degp = deg_call(col2, w2, zer1).reshape(NC, Np)
    dis2, y3 = _tc_prep(degp.reshape(NC, R, 128), xp3, R)
    y = y3.reshape(Np, Dp)
    dis1 = dis2.reshape(Np, 1)
    accp = agg1_call(row2, col2, w2, y, zer2).reshape(NC, Np, 16)
    zz = _tc_dense(accp[0], accp[1], y, dis1, W1p, b1r, W2, Np)
    acc2p = agg2_call(row2.reshape(Ep), col2, w2.reshape(Ep),
                      zz.reshape(Np), zer1).reshape(NC, Np)
    out2 = _tc_final(acc2p.reshape(NC, R, 128), zz.reshape(R, 128), dis2, b2r)
    return out2.reshape(Np)[:N].reshape(N, 1)
